# baseline (speedup 1.0000x reference)
import functools

import jax
import jax.numpy as jnp
from jax import lax
from jax.experimental import pallas as pl
from jax.experimental.pallas import tpu as pltpu
from jax.experimental.pallas import tpu_sc as plsc

_BLK = 256


def _build_repack(feat, vocab, nw, nc):
    nblocks = vocab // _BLK
    tail = vocab - nblocks * _BLK
    rounds = nblocks // nw
    rest = nblocks - rounds * nw
    mesh = plsc.VectorSubcoreMesh(core_axis_name="c", subcore_axis_name="s")

    @functools.partial(
        pl.kernel,
        mesh=mesh,
        out_type=jax.ShapeDtypeStruct((vocab, 128), jnp.float32),
        scratch_types=[
            pltpu.VMEM((2, feat, _BLK + 1), jnp.float32),
            pltpu.VMEM((2, _BLK, 128), jnp.float32),
            pltpu.SemaphoreType.DMA,
            pltpu.SemaphoreType.DMA,
        ],
        compiler_params=pltpu.CompilerParams(
            use_tc_tiling_on_sc=True, needs_layout_passes=False
        ),
    )
    def repack(tokt_hbm, tail_hbm, stage_hbm, blk, tblk, isem, osem):
        wid = lax.axis_index("s") * nc + lax.axis_index("c")
        iota = lax.iota(jnp.int32, 16)

        def start_in(r, h):
            base = pl.multiple_of((r * nw + wid) * _BLK, 128)
            pltpu.async_copy(
                tokt_hbm.at[:, pl.ds(base, _BLK)],
                blk.at[h, pl.ds(0, feat), pl.ds(0, _BLK)],
                isem,
            )

        def wait_in(h):
            pltpu.make_async_copy(
                tokt_hbm.at[:, pl.ds(0, _BLK)],
                blk.at[h, pl.ds(0, feat), pl.ds(0, _BLK)],
                isem,
            ).wait()

        def transpose(h):
            rows = [iota + 16 * j for j in range(feat // 16)]

            def trow(t, carry):
                tv = jnp.full((16,), t, jnp.int32)
                for j in range(feat // 16):
                    x = plsc.load_gather(blk.at[h], [rows[j], tv])
                    tblk[h, t, pl.ds(16 * j, 16)] = x
                return carry

            lax.fori_loop(0, _BLK, trow, 0)

        def start_out(r, h):
            base = pl.multiple_of((r * nw + wid) * _BLK, 128)
            pltpu.async_copy(tblk.at[h], stage_hbm.at[pl.ds(base, _BLK)], osem)

        def wait_out(h):
            pltpu.make_async_copy(
                stage_hbm.at[pl.ds(0, _BLK)], tblk.at[h], osem
            ).wait()

        start_in(0, 0)

        def outer(g, carry):
            for half in range(2):
                r = g * 2 + half
                wait_in(half)

                @pl.when(r + 1 < rounds)
                def _():
                    start_in(r + 1, 1 - half)

                @pl.when(r >= 2)
                def _():
                    wait_out(half)

                transpose(half)
                start_out(r, half)
            return carry

        lax.fori_loop(0, rounds // 2, outer, 0)
        wait_out(0)
        wait_out(1)

        if rest:
            @pl.when(wid < rest)
            def _():
                base = pl.multiple_of((rounds * nw + wid) * _BLK, 128)
                pltpu.sync_copy(
                    tokt_hbm.at[:, pl.ds(base, _BLK)],
                    blk.at[0, pl.ds(0, feat), pl.ds(0, _BLK)],
                )
                transpose(0)
                pltpu.sync_copy(tblk.at[0], stage_hbm.at[pl.ds(base, _BLK)])

        if tail:
            @pl.when(wid == nw - 1)
            def _():
                pltpu.sync_copy(
                    tail_hbm, stage_hbm.at[pl.ds(nblocks * _BLK, tail)]
                )

    return repack


def _build_lookup(seq, feat, vocab, nw, nc):
    mesh = plsc.VectorSubcoreMesh(core_axis_name="c", subcore_axis_name="s")

    @functools.partial(
        pl.kernel,
        mesh=mesh,
        out_type=jax.ShapeDtypeStruct((seq, feat, nw * 128), jnp.float32),
        scratch_types=[
            pltpu.VMEM((seq, 128), jnp.int32),
            pltpu.VMEM((seq, feat), jnp.float32),
            pltpu.VMEM((2, 128, 128), jnp.float32),
            pltpu.VMEM((2, feat, 129), jnp.float32),
            pltpu.SemaphoreType.DMA,
            pltpu.SemaphoreType.DMA,
        ],
        compiler_params=pltpu.CompilerParams(
            use_tc_tiling_on_sc=True, needs_layout_passes=False
        ),
    )
    def lookup(idxt_hbm, stage_hbm, pos_hbm, out_hbm, idx_v, pos_v, r2, tb, gsem, osem):
        wid = lax.axis_index("s") * nc + lax.axis_index("c")
        col0 = pl.multiple_of(wid * 128, 128)
        iota = lax.iota(jnp.int32, 16)
        pltpu.sync_copy(idxt_hbm.at[:, pl.ds(col0, 128)], idx_v)
        pltpu.sync_copy(pos_hbm, pos_v)

        def start_gather(s, h):
            pltpu.async_copy(stage_hbm.at[idx_v.at[s]], r2.at[h], gsem)

        def wait_gather(h):
            pltpu.make_async_copy(
                stage_hbm.at[pl.ds(0, 128)], r2.at[h], gsem
            ).wait()

        def start_out(s, h):
            pltpu.async_copy(
                tb.at[h, pl.ds(0, feat), pl.ds(0, 128)],
                out_hbm.at[s, pl.ds(0, feat), pl.ds(col0, 128)],
                osem,
            )

        def wait_out(h):
            pltpu.make_async_copy(
                out_hbm.at[0, pl.ds(0, feat), pl.ds(0, 128)],
                tb.at[h, pl.ds(0, feat), pl.ds(0, 128)],
                osem,
            ).wait()

        start_gather(0, 0)

        def outer(g, carry):
            for half in range(2):
                s = g * 2 + half
                wait_gather(half)

                @pl.when(s + 1 < seq)
                def _():
                    start_gather(s + 1, 1 - half)

                @pl.when(s >= 2)
                def _():
                    wait_out(half)

                pos_j = [pos_v[s, pl.ds(16 * j, 16)] for j in range(feat // 16)]

                def trow(t, inner):
                    tv = jnp.full((16,), t, jnp.int32)
                    for j in range(feat // 16):
                        x = r2[half, t, pl.ds(16 * j, 16)] + pos_j[j]
                        plsc.store_scatter(tb.at[half], [iota + 16 * j, tv], x)
                    return inner

                lax.fori_loop(0, 128, trow, 0)
                start_out(s, half)
            return carry

        lax.fori_loop(0, seq // 2, outer, 0)
        wait_out(0)
        wait_out(1)

    return lookup


def kernel(inputs, token_table, pos_table):
    batch, seq = inputs.shape
    vocab, feat = token_table.shape
    info = plsc.get_sparse_core_info()
    nc, ns = info.num_cores, info.num_subcores
    nw = nc * ns

    tok_t = token_table.T
    idx_t = inputs.astype(jnp.int32).T

    repack = _build_repack(feat, vocab, nw, nc)
    tail = vocab % _BLK
    tail_rows = jnp.pad(token_table[vocab - tail:], ((0, 0), (0, 128 - feat)))
    stage = repack(tok_t, tail_rows)

    lookup = _build_lookup(seq, feat, vocab, nw, nc)
    out_t = lookup(idx_t, stage, pos_table)
    return out_t.transpose(2, 0, 1)


# speedup vs baseline: 1.0096x; 1.0096x over previous
import functools

import jax
import jax.numpy as jnp
from jax import lax
from jax.experimental import pallas as pl
from jax.experimental.pallas import tpu as pltpu
from jax.experimental.pallas import tpu_sc as plsc

_BLK = 256


def _build_repack(feat, vocab, nw, nc):
    nblocks = vocab // _BLK
    tail = vocab - nblocks * _BLK
    rounds = nblocks // nw
    rest = nblocks - rounds * nw
    mesh = plsc.VectorSubcoreMesh(core_axis_name="c", subcore_axis_name="s")

    @functools.partial(
        pl.kernel,
        mesh=mesh,
        out_type=jax.ShapeDtypeStruct((vocab, 128), jnp.float32),
        scratch_types=[
            pltpu.VMEM((2, feat, _BLK + 1), jnp.float32),
            pltpu.VMEM((2, _BLK, 128), jnp.float32),
            pltpu.SemaphoreType.DMA,
            pltpu.SemaphoreType.DMA,
        ],
        compiler_params=pltpu.CompilerParams(
            use_tc_tiling_on_sc=True, needs_layout_passes=False
        ),
    )
    def repack(tokt_hbm, tail_hbm, stage_hbm, blk, tblk, isem, osem):
        wid = lax.axis_index("s") * nc + lax.axis_index("c")
        iota = lax.iota(jnp.int32, 16)

        def start_in(r, h):
            base = pl.multiple_of((r * nw + wid) * _BLK, 128)
            pltpu.async_copy(
                tokt_hbm.at[:, pl.ds(base, _BLK)],
                blk.at[h, pl.ds(0, feat), pl.ds(0, _BLK)],
                isem,
            )

        def wait_in(h):
            pltpu.make_async_copy(
                tokt_hbm.at[:, pl.ds(0, _BLK)],
                blk.at[h, pl.ds(0, feat), pl.ds(0, _BLK)],
                isem,
            ).wait()

        def transpose(h):
            rows = [iota + 16 * j for j in range(feat // 16)]

            def trow(t, carry):
                tv = jnp.full((16,), t, jnp.int32)
                for j in range(feat // 16):
                    x = plsc.load_gather(blk.at[h], [rows[j], tv])
                    tblk[h, t, pl.ds(16 * j, 16)] = x
                return carry

            lax.fori_loop(0, _BLK, trow, 0, unroll=8)

        def start_out(r, h):
            base = pl.multiple_of((r * nw + wid) * _BLK, 128)
            pltpu.async_copy(tblk.at[h], stage_hbm.at[pl.ds(base, _BLK)], osem)

        def wait_out(h):
            pltpu.make_async_copy(
                stage_hbm.at[pl.ds(0, _BLK)], tblk.at[h], osem
            ).wait()

        start_in(0, 0)

        def outer(g, carry):
            for half in range(2):
                r = g * 2 + half
                wait_in(half)

                @pl.when(r + 1 < rounds)
                def _():
                    start_in(r + 1, 1 - half)

                @pl.when(r >= 2)
                def _():
                    wait_out(half)

                transpose(half)
                start_out(r, half)
            return carry

        lax.fori_loop(0, rounds // 2, outer, 0)
        wait_out(0)
        wait_out(1)

        if rest:
            @pl.when(wid < rest)
            def _():
                base = pl.multiple_of((rounds * nw + wid) * _BLK, 128)
                pltpu.sync_copy(
                    tokt_hbm.at[:, pl.ds(base, _BLK)],
                    blk.at[0, pl.ds(0, feat), pl.ds(0, _BLK)],
                )
                transpose(0)
                pltpu.sync_copy(tblk.at[0], stage_hbm.at[pl.ds(base, _BLK)])

        if tail:
            @pl.when(wid == nw - 1)
            def _():
                pltpu.sync_copy(
                    tail_hbm, stage_hbm.at[pl.ds(nblocks * _BLK, tail)]
                )

    return repack


def _build_lookup(seq, feat, vocab, nw, nc):
    mesh = plsc.VectorSubcoreMesh(core_axis_name="c", subcore_axis_name="s")

    @functools.partial(
        pl.kernel,
        mesh=mesh,
        out_type=jax.ShapeDtypeStruct((seq, feat, nw * 128), jnp.float32),
        scratch_types=[
            pltpu.VMEM((seq, 128), jnp.int32),
            pltpu.VMEM((seq, feat), jnp.float32),
            pltpu.VMEM((2, 128, 128), jnp.float32),
            pltpu.VMEM((2, feat, 129), jnp.float32),
            pltpu.SemaphoreType.DMA,
            pltpu.SemaphoreType.DMA,
        ],
        compiler_params=pltpu.CompilerParams(
            use_tc_tiling_on_sc=True, needs_layout_passes=False
        ),
    )
    def lookup(idxt_hbm, stage_hbm, pos_hbm, out_hbm, idx_v, pos_v, r2, tb, gsem, osem):
        wid = lax.axis_index("s") * nc + lax.axis_index("c")
        col0 = pl.multiple_of(wid * 128, 128)
        iota = lax.iota(jnp.int32, 16)
        pltpu.sync_copy(idxt_hbm.at[:, pl.ds(col0, 128)], idx_v)
        pltpu.sync_copy(pos_hbm, pos_v)

        def start_gather(s, h):
            pltpu.async_copy(stage_hbm.at[idx_v.at[s]], r2.at[h], gsem)

        def wait_gather(h):
            pltpu.make_async_copy(
                stage_hbm.at[pl.ds(0, 128)], r2.at[h], gsem
            ).wait()

        def start_out(s, h):
            pltpu.async_copy(
                tb.at[h, pl.ds(0, feat), pl.ds(0, 128)],
                out_hbm.at[s, pl.ds(0, feat), pl.ds(col0, 128)],
                osem,
            )

        def wait_out(h):
            pltpu.make_async_copy(
                out_hbm.at[0, pl.ds(0, feat), pl.ds(0, 128)],
                tb.at[h, pl.ds(0, feat), pl.ds(0, 128)],
                osem,
            ).wait()

        start_gather(0, 0)

        def outer(g, carry):
            for half in range(2):
                s = g * 2 + half
                wait_gather(half)

                @pl.when(s + 1 < seq)
                def _():
                    start_gather(s + 1, 1 - half)

                @pl.when(s >= 2)
                def _():
                    wait_out(half)

                pos_j = [pos_v[s, pl.ds(16 * j, 16)] for j in range(feat // 16)]

                def trow(t, inner):
                    tv = jnp.full((16,), t, jnp.int32)
                    for j in range(feat // 16):
                        x = r2[half, t, pl.ds(16 * j, 16)] + pos_j[j]
                        plsc.store_scatter(tb.at[half], [iota + 16 * j, tv], x)
                    return inner

                lax.fori_loop(0, 128, trow, 0, unroll=8)
                start_out(s, half)
            return carry

        lax.fori_loop(0, seq // 2, outer, 0)
        wait_out(0)
        wait_out(1)

    return lookup


def kernel(inputs, token_table, pos_table):
    batch, seq = inputs.shape
    vocab, feat = token_table.shape
    info = plsc.get_sparse_core_info()
    nc, ns = info.num_cores, info.num_subcores
    nw = nc * ns

    tok_t = token_table.T
    idx_t = inputs.astype(jnp.int32).T

    repack = _build_repack(feat, vocab, nw, nc)
    tail = vocab % _BLK
    tail_rows = jnp.pad(token_table[vocab - tail:], ((0, 0), (0, 128 - feat)))
    stage = repack(tok_t, tail_rows)

    lookup = _build_lookup(seq, feat, vocab, nw, nc)
    out_t = lookup(idx_t, stage, pos_table)
    return out_t.transpose(2, 0, 1)


# speedup vs baseline: 2.1112x; 2.0912x over previous
import functools

import jax
import jax.numpy as jnp
from jax import lax
from jax.experimental import pallas as pl
from jax.experimental.pallas import tpu as pltpu
from jax.experimental.pallas import tpu_sc as plsc


_NBUF = 4


def _build(seq, feat, seqs_per_w, nw, nc):
    half = seq // 2
    nbuf = _NBUF
    mesh = plsc.VectorSubcoreMesh(core_axis_name="c", subcore_axis_name="s")

    @functools.partial(
        pl.kernel,
        mesh=mesh,
        out_type=jax.ShapeDtypeStruct((nw * seqs_per_w, seq, feat), jnp.float32),
        scratch_types=[
            pltpu.VMEM((seqs_per_w, 2, half), jnp.int32),
            pltpu.VMEM((seq, feat), jnp.float32),
            pltpu.VMEM((nbuf, seq, feat), jnp.float32),
            pltpu.SemaphoreType.DMA,
            pltpu.SemaphoreType.DMA,
        ],
        compiler_params=pltpu.CompilerParams(use_tc_tiling_on_sc=False),
    )
    def emb_kernel(idx_hbm, tok_hbm, pos_hbm, out_hbm, idx_v, pos_v, rows_v, gsem, osem):
        wid = lax.axis_index("s") * nc + lax.axis_index("c")
        pltpu.sync_copy(idx_hbm.at[wid], idx_v)
        pltpu.sync_copy(pos_hbm, pos_v)

        def start_gather(c, b):
            pltpu.async_copy(
                tok_hbm.at[idx_v.at[c, 0]], rows_v.at[b, pl.ds(0, half)], gsem
            )
            pltpu.async_copy(
                tok_hbm.at[idx_v.at[c, 1]], rows_v.at[b, pl.ds(half, half)], gsem
            )

        def wait_gather(b):
            pltpu.make_async_copy(
                tok_hbm.at[pl.ds(0, seq)], rows_v.at[b], gsem
            ).wait()

        def start_out(c, b):
            pltpu.async_copy(rows_v.at[b], out_hbm.at[wid * seqs_per_w + c], osem)

        def wait_out(b):
            pltpu.make_async_copy(
                out_hbm.at[0], rows_v.at[b], osem
            ).wait()

        start_gather(0, 0)
        start_gather(1, 1)

        def outer(g, carry):
            for b in range(nbuf):
                c = g * nbuf + b
                wait_gather(b)

                def srow(s, inner):
                    for j in range(feat // 16):
                        sl = pl.ds(j * 16, 16)
                        plsc.addupdate(rows_v.at[b, s, sl], pos_v[s, sl])
                    return inner

                lax.fori_loop(0, seq, srow, 0)
                start_out(c, b)

                bn = (b + 2) % nbuf

                @pl.when(c >= 2)
                def _():
                    wait_out(bn)

                @pl.when(c + 2 < seqs_per_w)
                def _():
                    start_gather(c + 2, bn)

            return carry

        lax.fori_loop(0, seqs_per_w // nbuf, outer, 0)
        wait_out((seqs_per_w - 2) % nbuf)
        wait_out((seqs_per_w - 1) % nbuf)

    return emb_kernel


def kernel(inputs, token_table, pos_table):
    batch, seq = inputs.shape
    feat = token_table.shape[1]
    info = plsc.get_sparse_core_info()
    nc, ns = info.num_cores, info.num_subcores
    nw = nc * ns
    total = batch * seq
    rows_per_w = total // nw
    seqs_per_w = rows_per_w // seq

    emb = _build(seq, feat, seqs_per_w, nw, nc)
    idx = inputs.astype(jnp.int32).reshape(nw, seqs_per_w, 2, seq // 2)
    out = emb(idx, token_table, pos_table)
    return out
